# baseline (speedup 1.0000x reference)
import functools

import jax
import jax.numpy as jnp
from jax.experimental import pallas as pl
from jax.experimental.pallas import tpu as pltpu

F32 = jnp.float32
HIGHEST = jax.lax.Precision.HIGHEST


def _dot(a, b):
    return jax.lax.dot_general(a, b, (((1,), (0,)), ((), ())),
                               precision=HIGHEST, preferred_element_type=F32)


def _t0_body(x_ref, w_ref, b_ref, o_ref):
    o_ref[...] = jax.nn.relu(_dot(x_ref[...], w_ref[...]) + b_ref[...])


def _t0(x, w, b, tile):
    n, d = x.shape
    k = w.shape[1]
    return pl.pallas_call(
        _t0_body,
        grid=(n // tile,),
        in_specs=[
            pl.BlockSpec((tile, d), lambda i: (i, 0)),
            pl.BlockSpec((d, k), lambda i: (0, 0)),
            pl.BlockSpec((1, k), lambda i: (0, 0)),
        ],
        out_specs=pl.BlockSpec((tile, k), lambda i: (i, 0)),
        out_shape=jax.ShapeDtypeStruct((n, k), F32),
    )(x, w, b.reshape(1, k))


def _t1_body(ea_ref, hs_ref, w1_ref, b1_ref, w2_ref, b2_ref, o_ref):
    te = ea_ref.shape[0]
    a = jax.nn.relu(_dot(ea_ref[...], w1_ref[...]) + b1_ref[...])
    we = _dot(a, w2_ref[...]) + b2_ref[...]
    iot = jax.lax.broadcasted_iota(jnp.int32, (16, 256), 1)
    expand = (iot // 16 == jax.lax.broadcasted_iota(jnp.int32, (16, 256), 0)
              ).astype(F32)
    hs_exp = _dot(hs_ref[...], expand)
    v = hs_exp * we
    iot2 = jax.lax.broadcasted_iota(jnp.int32, (256, 16), 0)
    red = (iot2 % 16 == jax.lax.broadcasted_iota(jnp.int32, (256, 16), 1)
           ).astype(F32)
    o_ref[...] = _dot(v, red)


def _t1(ea, hs, w1, b1, w2, b2, tile):
    e, bd = ea.shape
    internal = w1.shape[1]
    io2 = w2.shape[1]
    in_dim = hs.shape[1]
    return pl.pallas_call(
        _t1_body,
        grid=(e // tile,),
        in_specs=[
            pl.BlockSpec((tile, bd), lambda i: (i, 0)),
            pl.BlockSpec((tile, in_dim), lambda i: (i, 0)),
            pl.BlockSpec((bd, internal), lambda i: (0, 0)),
            pl.BlockSpec((1, internal), lambda i: (0, 0)),
            pl.BlockSpec((internal, io2), lambda i: (0, 0)),
            pl.BlockSpec((1, io2), lambda i: (0, 0)),
        ],
        out_specs=pl.BlockSpec((tile, in_dim), lambda i: (i, 0)),
        out_shape=jax.ShapeDtypeStruct((e, in_dim), F32),
    )(ea, hs, w1, b1.reshape(1, internal), w2, b2.reshape(1, io2))


def _t2_body(agg_ref, rdeg_ref, h_ref, bt_ref, root_ref, cb_ref, wfp_ref,
             bfp_ref, hn_ref, pooled_ref, cnt_ref, *, num_graphs):
    i = pl.program_id(0)
    hn = jax.nn.relu(agg_ref[...] * rdeg_ref[...]
                     + _dot(h_ref[...], root_ref[...]) + cb_ref[...])
    hn_ref[...] = hn
    ex = jax.nn.relu(_dot(hn, wfp_ref[...]) + bfp_ref[...])
    onehot = (bt_ref[...] == jax.lax.broadcasted_iota(
        jnp.int32, (1, num_graphs), 1)).astype(F32)
    pooled = jax.lax.dot_general(onehot, ex, (((0,), (0,)), ((), ())),
                                 precision=HIGHEST, preferred_element_type=F32)
    cnt = jnp.sum(onehot, axis=0, keepdims=True)

    @pl.when(i == 0)
    def _():
        pooled_ref[...] = jnp.zeros_like(pooled_ref)
        cnt_ref[...] = jnp.zeros_like(cnt_ref)

    pooled_ref[...] += pooled
    cnt_ref[...] += cnt


def _t2(agg, rdeg, h, batch_col, root, cb, wfp, bfp, num_graphs, tile):
    n, in_dim = h.shape
    fp_dim = wfp.shape[1]
    body = functools.partial(_t2_body, num_graphs=num_graphs)
    return pl.pallas_call(
        body,
        grid=(n // tile,),
        in_specs=[
            pl.BlockSpec((tile, in_dim), lambda i: (i, 0)),
            pl.BlockSpec((tile, 1), lambda i: (i, 0)),
            pl.BlockSpec((tile, in_dim), lambda i: (i, 0)),
            pl.BlockSpec((tile, 1), lambda i: (i, 0)),
            pl.BlockSpec((in_dim, in_dim), lambda i: (0, 0)),
            pl.BlockSpec((1, in_dim), lambda i: (0, 0)),
            pl.BlockSpec((in_dim, fp_dim), lambda i: (0, 0)),
            pl.BlockSpec((1, fp_dim), lambda i: (0, 0)),
        ],
        out_specs=[
            pl.BlockSpec((tile, in_dim), lambda i: (i, 0)),
            pl.BlockSpec((num_graphs, fp_dim), lambda i: (0, 0)),
            pl.BlockSpec((1, num_graphs), lambda i: (0, 0)),
        ],
        out_shape=[
            jax.ShapeDtypeStruct((n, in_dim), F32),
            jax.ShapeDtypeStruct((num_graphs, fp_dim), F32),
            jax.ShapeDtypeStruct((1, num_graphs), F32),
        ],
    )(agg, rdeg, h, batch_col, root, cb.reshape(1, in_dim), wfp,
      bfp.reshape(1, fp_dim))


def kernel(x, edge_index, edge_attr, batch, W_atom, b_atom, W_e1, b_e1,
           W_e2, b_e2, root, conv_bias, W_fp, b_fp):
    n_nodes, node_dim = x.shape
    n_edges, bond_dim = edge_attr.shape
    in_dim = root.shape[0]
    fp_dim = W_fp.shape[1]
    depth = 3
    num_graphs = 64

    src = edge_index[0].astype(jnp.int32)
    dst = edge_index[1].astype(jnp.int32)
    batch32 = batch.astype(jnp.int32).reshape(n_nodes, 1)

    h = _t0(x, W_atom, b_atom, tile=1000)

    deg = jax.ops.segment_sum(jnp.ones((n_edges,), F32), dst,
                              num_segments=n_nodes)
    rdeg = (1.0 / jnp.maximum(deg, 1.0)).reshape(n_nodes, 1)

    pooled_sum = jnp.zeros((num_graphs, fp_dim), F32)
    counts = None
    for _ in range(depth):
        hs = jnp.take(h, src, axis=0)
        msg = _t1(edge_attr, hs, W_e1, b_e1, W_e2, b_e2, tile=512)
        agg = jax.ops.segment_sum(msg, dst, num_segments=n_nodes)
        h, pooled, cnt = _t2(agg, rdeg, h, batch32, root, conv_bias,
                             W_fp, b_fp, num_graphs, tile=1000)
        pooled_sum = pooled_sum + pooled
        counts = cnt

    counts = jnp.maximum(counts, 1.0)
    return pooled_sum / counts.reshape(num_graphs, 1)
